# Initial kernel scaffold; baseline (speedup 1.0000x reference)
import jax
import jax.numpy as jnp
from jax.experimental import pallas as pl


def kernel(generated_tokens, generated_logits, vocab_size):
    raise NotImplementedError("write your pallas kernel here")


# speedup vs baseline: 3.1663x; 3.1663x over previous
import functools

import jax
import jax.numpy as jnp
import numpy as np
from jax import lax
from jax.experimental import pallas as pl
from jax.experimental.pallas import tpu as pltpu

_B, _S, _V = 32, 200, 1000
_NBI = _S - 1
_NTRI = _S - 2


def _conf_body(lg_ref, out_ref):
    i = pl.program_id(0)
    x = lg_ref[...]
    m = jnp.max(x, axis=1, keepdims=True)
    s = jnp.sum(jnp.exp(x - m), axis=1)
    part = jnp.sum(1.0 / s)

    @pl.when(i == 0)
    def _():
        out_ref[...] = jnp.zeros((1, 1), jnp.float32)

    out_ref[...] += jnp.full((1, 1), part)


def _stats_body(toks_ref, conf_ref, out_ref, counts_ref, pres_ref, bi_ref, tri_ref):
    toks = toks_ref[...]
    bi = toks[:, :-1] * _V + toks[:, 1:]
    tri = bi[:, :-1] * _V + toks[:, 2:]
    bi_ref[...] = bi
    tri_ref[...] = tri

    e3 = tri[:, :, None] == tri[:, None, :]
    i_s = lax.broadcasted_iota(jnp.int32, (1, _NTRI, _NTRI), 1)
    i_sp = lax.broadcasted_iota(jnp.int32, (1, _NTRI, _NTRI), 2)
    dup = jnp.any(e3 & (i_sp < i_s), axis=2)
    uniq_rows = _NTRI - jnp.sum(dup.astype(jnp.float32), axis=1)
    repetition = jnp.mean(1.0 - uniq_rows / _NTRI)

    counts_ref[...] = jnp.zeros((1, _V), jnp.float32)
    iota_v = lax.broadcasted_iota(jnp.int32, (1, _V), 1)

    def hist_body(b, _):
        row = toks_ref[b, :]
        cmp = row[:, None] == iota_v
        counts_ref[...] += jnp.sum(cmp.astype(jnp.float32), axis=0)[None, :]
        pres_ref[pl.ds(b, 1), :] = jnp.any(cmp, axis=0).astype(jnp.float32)[None, :]
        return 0

    lax.fori_loop(0, _B, hist_body, 0)

    counts = counts_ref[0, :]
    total = jnp.sum(counts)
    probs = counts / (total + 1e-08)
    entropy = -jnp.sum(jnp.where(probs > 0, probs * jnp.log(probs + 1e-08), 0.0))
    token_entropy = 1.0 - entropy / np.log(_V)
    distinct1 = jnp.sum((counts > 0).astype(jnp.float32))

    pres = pres_ref[...]
    inter = lax.dot_general(pres, pres, (((1,), (1,)), ((), ())),
                            preferred_element_type=jnp.float32)
    ru = jnp.sum(pres, axis=1)
    r_i = lax.broadcasted_iota(jnp.int32, (_B, _B), 0)
    c_i = lax.broadcasted_iota(jnp.int32, (_B, _B), 1)
    selmask = ((r_i < 10) & (r_i != c_i)).astype(jnp.float32)
    overlaps = inter / jnp.maximum(ru, 1.0)[:, None]
    self_bleu = jnp.sum(overlaps * selmask) / (10 * (_B - 1))

    ib_b = lax.broadcasted_iota(jnp.int32, (_NBI, _B, _NBI), 1)
    ib_sp = lax.broadcasted_iota(jnp.int32, (_NBI, _B, _NBI), 2)
    ib_s = lax.broadcasted_iota(jnp.int32, (_NBI, _B, _NBI), 0)
    gpos_bi = ib_b * _NBI + ib_sp
    it_b = lax.broadcasted_iota(jnp.int32, (_NTRI, _B, _NTRI), 1)
    it_sp = lax.broadcasted_iota(jnp.int32, (_NTRI, _B, _NTRI), 2)
    it_s = lax.broadcasted_iota(jnp.int32, (_NTRI, _B, _NTRI), 0)
    gpos_tri = it_b * _NTRI + it_sp

    def uniq_body(b, carry):
        ubi, utri = carry
        rowb = bi_ref[b, :]
        eq = rowb[:, None, None] == bi[None, :, :]
        earlier = gpos_bi < (b * _NBI + ib_s)
        dup_b = jnp.any(eq & earlier, axis=2)
        dup_b = jnp.any(dup_b, axis=1)
        ubi += jnp.sum((~dup_b).astype(jnp.float32))
        rowt = tri_ref[b, :]
        eqt = rowt[:, None, None] == tri[None, :, :]
        earliert = gpos_tri < (b * _NTRI + it_s)
        dup_t = jnp.any(eqt & earliert, axis=2)
        dup_t = jnp.any(dup_t, axis=1)
        utri += jnp.sum((~dup_t).astype(jnp.float32))
        return (ubi, utri)

    u_bi, u_tri = lax.fori_loop(0, _B, uniq_body, (jnp.float32(0), jnp.float32(0)))

    d1 = distinct1 / (_B * _S)
    d2 = u_bi / (_B * _NBI)
    d3 = u_tri / (_B * _NTRI)
    ngram_diversity = ((1.0 - d1) + (1.0 - d2) + (1.0 - d3)) / 3.0

    avg_conf = jnp.sum(conf_ref[...]) / (_B * _S)
    overconfidence = jnp.maximum(avg_conf - 0.85, 0.0) * 2.0

    total_loss = (0.25 * ngram_diversity + 0.2 * token_entropy + 0.2 * self_bleu
                  + 0.2 * repetition + 0.15 * overconfidence)

    out_ref[...] = jnp.stack([ngram_diversity, token_entropy, self_bleu,
                              repetition, overconfidence, total_loss])[None, :]


@jax.jit
def _run(toks, logits):
    lg2 = logits.reshape(_B * _S, _V)
    rows = 800
    conf = pl.pallas_call(
        _conf_body,
        grid=(_B * _S // rows,),
        in_specs=[pl.BlockSpec((rows, _V), lambda i: (i, 0))],
        out_specs=pl.BlockSpec((1, 1), lambda i: (0, 0)),
        out_shape=jax.ShapeDtypeStruct((1, 1), jnp.float32),
    )(lg2)

    out = pl.pallas_call(
        _stats_body,
        out_shape=jax.ShapeDtypeStruct((1, 6), jnp.float32),
        scratch_shapes=[
            pltpu.VMEM((1, _V), jnp.float32),
            pltpu.VMEM((_B, _V), jnp.float32),
            pltpu.VMEM((_B, _NBI), jnp.int32),
            pltpu.VMEM((_B, _NTRI), jnp.int32),
        ],
    )(toks.astype(jnp.int32), conf)
    return out.reshape(6)


def kernel(generated_tokens, generated_logits, vocab_size):
    return _run(generated_tokens, generated_logits)
